# baseline (speedup 1.0000x reference)
import jax
import jax.numpy as jnp
from jax.experimental import pallas as pl
from jax.experimental.pallas import tpu as pltpu

_NUM_LOGPROBS = 5
_NEG_INF = -1e9
_R = 8


def _fold(a, op, width):
    V = a.shape[-1]
    n = V // width
    f = a[:, :width]
    for i in range(1, n):
        f = op(f, a[:, i * width:(i + 1) * width])
    return f


def _rmax(a):
    f = _fold(a, jnp.maximum, 10000)
    f = _fold(f, jnp.maximum, 1000)
    return jnp.max(f, axis=-1, keepdims=True)


def _rmin(a):
    f = _fold(a, jnp.minimum, 10000)
    f = _fold(f, jnp.minimum, 1000)
    return jnp.min(f, axis=-1, keepdims=True)


def _rsum(a):
    f = _fold(a, jnp.add, 10000)
    f = _fold(f, jnp.add, 1000)
    return jnp.sum(f, axis=-1, keepdims=True)


def _sampler_block(l_ref, g_ref, t_ref, p_ref, k_ref,
                   sid_ref, tkl_ref, tki_ref, slp_ref, *, K):
    l = l_ref[...]
    R, V = l.shape
    t = jnp.maximum(t_ref[...], 1e-5)
    p = p_ref[...]
    del k_ref

    m = _rmax(l)
    se = _rsum(jnp.exp(l - m))
    lse = m + jnp.log(se)

    iota = jax.lax.broadcasted_iota(jnp.int32, (R, V), 1)
    iota_k = jax.lax.broadcasted_iota(jnp.int32, (R, K), 1)

    def body(i, carry):
        work, vals, ids = carry
        mv = _rmax(work)
        idx = _rmin(jnp.where(work == mv, iota, V))
        sel = iota_k == i
        vals = jnp.where(sel, mv, vals)
        ids = jnp.where(sel, idx, ids)
        work = jnp.where(iota == idx, _NEG_INF, work)
        return work, vals, ids

    vals0 = jnp.full((R, K), _NEG_INF, dtype=l.dtype)
    ids0 = jnp.zeros((R, K), dtype=jnp.int32)
    _, vals, ids = jax.lax.fori_loop(0, K, body, (l, vals0, ids0))

    x50 = vals / t
    ex = jnp.exp(x50 - x50[:, :1])
    denom = jnp.sum(ex, axis=-1, keepdims=True)
    probs = ex / denom
    run = jnp.zeros((R, 1), dtype=l.dtype)
    exc_cols = []
    for j in range(K):
        exc_cols.append(run)
        run = run + probs[:, j:j + 1]
    exc = jnp.concatenate(exc_cols, axis=1)
    keep = exc <= p
    cutoff = jnp.min(jnp.where(keep, x50, jnp.inf), axis=-1, keepdims=True)

    x = l / t
    y = jnp.where(x >= cutoff, x, _NEG_INF) + g_ref[...]
    my = _rmax(y)
    sidx = _rmin(jnp.where(y == my, iota, V))

    sid_ref[...] = sidx
    sl = jnp.sum(jnp.where(ids == sidx, vals, 0.0), axis=-1, keepdims=True)
    slp_ref[...] = sl - lse
    tkl_ref[...] = vals[:, :_NUM_LOGPROBS] - lse
    tki_ref[...] = ids[:, :_NUM_LOGPROBS]


def kernel(logits, temperature, top_p, top_k):
    logits = logits.astype(jnp.float32)
    B, V = logits.shape
    try:
        K = int(top_k)
    except Exception:
        K = 50

    g = jax.random.gumbel(jax.random.key(12345), (B, V), dtype=jnp.float32)
    t2 = temperature.astype(jnp.float32).reshape(B, 1)
    p2 = top_p.astype(jnp.float32).reshape(B, 1)

    nblk = B // _R
    grid = (nblk,)
    row_spec = pl.BlockSpec((_R, V), lambda i: (i, 0))

    import functools
    body = functools.partial(_sampler_block, K=K)
    sid, tkl, tki, slp = pl.pallas_call(
        body,
        grid=grid,
        in_specs=[
            row_spec,
            row_spec,
            pl.BlockSpec((_R, 1), lambda i: (i, 0)),
            pl.BlockSpec((_R, 1), lambda i: (i, 0)),
            pl.BlockSpec((_R, 1), lambda i: (i, 0)),
        ],
        out_specs=[
            pl.BlockSpec((_R, 1), lambda i: (i, 0)),
            pl.BlockSpec((_R, _NUM_LOGPROBS), lambda i: (i, 0)),
            pl.BlockSpec((_R, _NUM_LOGPROBS), lambda i: (i, 0)),
            pl.BlockSpec((_R, 1), lambda i: (i, 0)),
        ],
        out_shape=[
            jax.ShapeDtypeStruct((B, 1), jnp.int32),
            jax.ShapeDtypeStruct((B, _NUM_LOGPROBS), jnp.float32),
            jax.ShapeDtypeStruct((B, _NUM_LOGPROBS), jnp.int32),
            jax.ShapeDtypeStruct((B, 1), jnp.float32),
        ],
        compiler_params=pltpu.CompilerParams(
            dimension_semantics=("parallel",)),
    )(logits, g, t2, p2,
      jnp.broadcast_to(jnp.asarray(top_k, jnp.int32).reshape(1, 1), (B, 1)))

    return (sid,
            tkl,
            tki.astype(jnp.int64),
            slp)


# speedup vs baseline: 1.4659x; 1.4659x over previous
import jax
import jax.numpy as jnp
from jax.experimental import pallas as pl
from jax.experimental.pallas import tpu as pltpu

_NUM_LOGPROBS = 5
_NEG_INF = -1e9
_R = 8


def _rmax(a):
    return jnp.max(a, axis=-1, keepdims=True)


def _rsum(a):
    return jnp.sum(a, axis=-1, keepdims=True)


def _argmax(a):
    return jnp.argmax(a, axis=-1).astype(jnp.int32)[:, None]


def _sampler_block(l_ref, g_ref, t_ref, p_ref, k_ref,
                   sid_ref, tkl_ref, tki_ref, slp_ref, *, K):
    l = l_ref[...]
    R, V = l.shape
    t = jnp.maximum(t_ref[...], 1e-5)
    p = p_ref[...]
    del k_ref

    m = _rmax(l)
    se = _rsum(jnp.exp(l - m))
    lse = m + jnp.log(se)

    iota = jax.lax.broadcasted_iota(jnp.int32, (R, V), 1)
    iota_k = jax.lax.broadcasted_iota(jnp.int32, (R, K), 1)

    def body(i, carry):
        work, vals, ids = carry
        mv = _rmax(work)
        idx = _argmax(work)
        sel = iota_k == i
        vals = jnp.where(sel, mv, vals)
        ids = jnp.where(sel, idx, ids)
        work = jnp.where(iota == idx, _NEG_INF, work)
        return work, vals, ids

    vals0 = jnp.full((R, K), _NEG_INF, dtype=l.dtype)
    ids0 = jnp.zeros((R, K), dtype=jnp.int32)
    _, vals, ids = jax.lax.fori_loop(0, K, body, (l, vals0, ids0))

    x50 = vals / t
    ex = jnp.exp(x50 - x50[:, :1])
    denom = jnp.sum(ex, axis=-1, keepdims=True)
    probs = ex / denom
    run = jnp.zeros((R, 1), dtype=l.dtype)
    exc_cols = []
    for j in range(K):
        exc_cols.append(run)
        run = run + probs[:, j:j + 1]
    exc = jnp.concatenate(exc_cols, axis=1)
    keep = exc <= p
    cutoff = jnp.min(jnp.where(keep, x50, jnp.inf), axis=-1, keepdims=True)

    x = l / t
    y = jnp.where(x >= cutoff, x, _NEG_INF) + g_ref[...]
    sidx = _argmax(y)

    sid_ref[...] = sidx
    sl = jnp.sum(jnp.where(ids == sidx, vals, 0.0), axis=-1, keepdims=True)
    slp_ref[...] = sl - lse
    tkl_ref[...] = vals[:, :_NUM_LOGPROBS] - lse
    tki_ref[...] = ids[:, :_NUM_LOGPROBS]


def kernel(logits, temperature, top_p, top_k):
    logits = logits.astype(jnp.float32)
    B, V = logits.shape
    try:
        K = int(top_k)
    except Exception:
        K = 50

    g = jax.random.gumbel(jax.random.key(12345), (B, V), dtype=jnp.float32)
    t2 = temperature.astype(jnp.float32).reshape(B, 1)
    p2 = top_p.astype(jnp.float32).reshape(B, 1)

    nblk = B // _R
    grid = (nblk,)
    row_spec = pl.BlockSpec((_R, V), lambda i: (i, 0))

    import functools
    body = functools.partial(_sampler_block, K=K)
    sid, tkl, tki, slp = pl.pallas_call(
        body,
        grid=grid,
        in_specs=[
            row_spec,
            row_spec,
            pl.BlockSpec((_R, 1), lambda i: (i, 0)),
            pl.BlockSpec((_R, 1), lambda i: (i, 0)),
            pl.BlockSpec((_R, 1), lambda i: (i, 0)),
        ],
        out_specs=[
            pl.BlockSpec((_R, 1), lambda i: (i, 0)),
            pl.BlockSpec((_R, _NUM_LOGPROBS), lambda i: (i, 0)),
            pl.BlockSpec((_R, _NUM_LOGPROBS), lambda i: (i, 0)),
            pl.BlockSpec((_R, 1), lambda i: (i, 0)),
        ],
        out_shape=[
            jax.ShapeDtypeStruct((B, 1), jnp.int32),
            jax.ShapeDtypeStruct((B, _NUM_LOGPROBS), jnp.float32),
            jax.ShapeDtypeStruct((B, _NUM_LOGPROBS), jnp.int32),
            jax.ShapeDtypeStruct((B, 1), jnp.float32),
        ],
        compiler_params=pltpu.CompilerParams(
            dimension_semantics=("parallel",)),
    )(logits, g, t2, p2,
      jnp.broadcast_to(jnp.asarray(top_k, jnp.int32).reshape(1, 1), (B, 1)))

    return (sid,
            tkl,
            tki.astype(jnp.int64),
            slp)


# speedup vs baseline: 1.5850x; 1.0813x over previous
import jax
import jax.numpy as jnp
from jax.experimental import pallas as pl
from jax.experimental.pallas import tpu as pltpu

_NUM_LOGPROBS = 5
_NEG_INF = -1e9
_R = 16


def _rmax(a):
    return jnp.max(a, axis=-1, keepdims=True)


def _rsum(a):
    return jnp.sum(a, axis=-1, keepdims=True)


def _argmax(a):
    return jnp.argmax(a, axis=-1).astype(jnp.int32)[:, None]


def _sampler_block(l_ref, g_ref, t_ref, p_ref, k_ref,
                   sid_ref, tkl_ref, tki_ref, slp_ref, *, K):
    l = l_ref[...]
    R, V = l.shape
    t = jnp.maximum(t_ref[...], 1e-5)
    p = p_ref[...]
    del k_ref

    m = _rmax(l)
    se = _rsum(jnp.exp(l - m))
    lse = m + jnp.log(se)

    iota = jax.lax.broadcasted_iota(jnp.int32, (R, V), 1)
    iota_k = jax.lax.broadcasted_iota(jnp.int32, (R, K), 1)

    def body(i, carry):
        work, vals, ids = carry
        mv = _rmax(work)
        idx = _argmax(work)
        sel = iota_k == i
        vals = jnp.where(sel, mv, vals)
        ids = jnp.where(sel, idx, ids)
        work = jnp.where(iota == idx, _NEG_INF, work)
        return work, vals, ids

    vals0 = jnp.full((R, K), _NEG_INF, dtype=l.dtype)
    ids0 = jnp.zeros((R, K), dtype=jnp.int32)
    _, vals, ids = jax.lax.fori_loop(0, K, body, (l, vals0, ids0))

    x50 = vals / t
    ex = jnp.exp(x50 - x50[:, :1])
    denom = jnp.sum(ex, axis=-1, keepdims=True)
    probs = ex / denom
    run = jnp.zeros((R, 1), dtype=l.dtype)
    exc_cols = []
    for j in range(K):
        exc_cols.append(run)
        run = run + probs[:, j:j + 1]
    exc = jnp.concatenate(exc_cols, axis=1)
    keep = exc <= p
    cutoff = jnp.min(jnp.where(keep, x50, jnp.inf), axis=-1, keepdims=True)

    x = l / t
    y = jnp.where(x >= cutoff, x, _NEG_INF) + g_ref[...]
    sidx = _argmax(y)

    sid_ref[...] = sidx
    sl = jnp.sum(jnp.where(ids == sidx, vals, 0.0), axis=-1, keepdims=True)
    slp_ref[...] = sl - lse
    tkl_ref[...] = vals[:, :_NUM_LOGPROBS] - lse
    tki_ref[...] = ids[:, :_NUM_LOGPROBS]


def kernel(logits, temperature, top_p, top_k):
    logits = logits.astype(jnp.float32)
    B, V = logits.shape
    try:
        K = int(top_k)
    except Exception:
        K = 50

    g = jax.random.gumbel(jax.random.key(12345), (B, V), dtype=jnp.float32)
    t2 = temperature.astype(jnp.float32).reshape(B, 1)
    p2 = top_p.astype(jnp.float32).reshape(B, 1)

    nblk = B // _R
    grid = (nblk,)
    row_spec = pl.BlockSpec((_R, V), lambda i: (i, 0))

    import functools
    body = functools.partial(_sampler_block, K=K)
    sid, tkl, tki, slp = pl.pallas_call(
        body,
        grid=grid,
        in_specs=[
            row_spec,
            row_spec,
            pl.BlockSpec((_R, 1), lambda i: (i, 0)),
            pl.BlockSpec((_R, 1), lambda i: (i, 0)),
            pl.BlockSpec((_R, 1), lambda i: (i, 0)),
        ],
        out_specs=[
            pl.BlockSpec((_R, 1), lambda i: (i, 0)),
            pl.BlockSpec((_R, _NUM_LOGPROBS), lambda i: (i, 0)),
            pl.BlockSpec((_R, _NUM_LOGPROBS), lambda i: (i, 0)),
            pl.BlockSpec((_R, 1), lambda i: (i, 0)),
        ],
        out_shape=[
            jax.ShapeDtypeStruct((B, 1), jnp.int32),
            jax.ShapeDtypeStruct((B, _NUM_LOGPROBS), jnp.float32),
            jax.ShapeDtypeStruct((B, _NUM_LOGPROBS), jnp.int32),
            jax.ShapeDtypeStruct((B, 1), jnp.float32),
        ],
        compiler_params=pltpu.CompilerParams(
            dimension_semantics=("parallel",)),
    )(logits, g, t2, p2,
      jnp.broadcast_to(jnp.asarray(top_k, jnp.int32).reshape(1, 1), (B, 1)))

    return (sid,
            tkl,
            tki.astype(jnp.int64),
            slp)
